# baseline (speedup 1.0000x reference)
import functools

import jax
import jax.numpy as jnp
from jax import lax
from jax.experimental import pallas as pl
from jax.experimental.pallas import tpu as pltpu
from jax.experimental.pallas import tpu_sc as plsc

NUM_EMB = 1024
DIM = 64
N_TOKENS = 16 * 64 * 64
BN = 512
N_CHUNKS = 4
CHUNK = N_TOKENS // N_CHUNKS
GRID = CHUNK // BN

NC, NS = 2, 16
NW = NC * NS
B_PER_W = CHUNK // NW
CH = 128


def _dist_argmin_body(x_ref, w_ref, idx_ref, w2_ref, iotaf_ref, wm2_ref):
    @pl.when(pl.program_id(0) == 0)
    def _():
        w = w_ref[...]
        w2_ref[...] = jnp.sum(w * w, axis=1)[None, :]
        ii = lax.broadcasted_iota(jnp.int32, (1, NUM_EMB), 1)
        iotaf_ref[...] = ii.astype(jnp.float32)
        wm2_ref[...] = w * -2.0

    x = x_ref[...]
    x2 = jnp.sum(x * x, axis=1, keepdims=True)
    mm2 = lax.dot_general(x, wm2_ref[...], (((1,), (1,)), ((), ())),
                          preferred_element_type=jnp.float32)
    d = (x2 + w2_ref[...]) + mm2
    dmin = jnp.min(d, axis=1, keepdims=True)
    idxf = jnp.min(jnp.where(d == dmin, iotaf_ref[...], float(2 * NUM_EMB)),
                   axis=1)
    idx_ref[0, 0, :] = idxf.astype(jnp.int32)


def _encode(x_flat, W):
    return pl.pallas_call(
        _dist_argmin_body,
        grid=(GRID,),
        in_specs=[
            pl.BlockSpec((BN, DIM), lambda i: (i, 0)),
            pl.BlockSpec((NUM_EMB, DIM), lambda i: (0, 0)),
        ],
        out_specs=pl.BlockSpec((1, 1, BN), lambda i: (i, 0, 0)),
        out_shape=jax.ShapeDtypeStruct((GRID, 1, BN), jnp.int32),
        scratch_shapes=[pltpu.VMEM((1, NUM_EMB), jnp.float32),
                        pltpu.VMEM((1, NUM_EMB), jnp.float32),
                        pltpu.VMEM((NUM_EMB, DIM), jnp.float32)],
    )(x_flat, W)


@functools.partial(
    pl.kernel,
    out_type=jax.ShapeDtypeStruct((CHUNK, DIM), jnp.float32),
    mesh=plsc.VectorSubcoreMesh(core_axis_name="c", subcore_axis_name="s"),
    compiler_params=pltpu.CompilerParams(use_tc_tiling_on_sc=False),
    scratch_types=[
        pltpu.VMEM((B_PER_W,), jnp.int32),
        pltpu.VMEM((CH, DIM), jnp.float32),
        pltpu.SemaphoreType.DMA,
    ],
)
def _gather_rows(w_hbm, idx_hbm, out_hbm, idx_v, rows_v, sem):
    wid = lax.axis_index("s") * NC + lax.axis_index("c")
    base = wid * B_PER_W
    pltpu.sync_copy(idx_hbm.at[pl.ds(base, B_PER_W)], idx_v)
    for c in range(B_PER_W // CH):
        pltpu.async_copy(
            w_hbm.at[idx_v.at[pl.ds(c * CH, CH)]], rows_v, sem).wait()
        pltpu.sync_copy(rows_v, out_hbm.at[pl.ds(base + c * CH, CH)])


def kernel(inputs, W):
    x = jnp.transpose(inputs, (0, 2, 3, 1))
    input_shape = x.shape
    x_flat = x.reshape(-1, DIM)
    idxs, quants = [], []
    for c in range(N_CHUNKS):
        xc = lax.slice_in_dim(x_flat, c * CHUNK, (c + 1) * CHUNK, axis=0)
        idx_c = _encode(xc, W).reshape(-1)
        idxs.append(idx_c)
        quants.append(_gather_rows(W, idx_c))
    idx = jnp.concatenate(idxs)
    quant_flat = jnp.concatenate(quants, axis=0)
    quantized = quant_flat.reshape(input_shape)
    quantized = jnp.transpose(quantized, (0, 3, 1, 2))
    return quantized, idx.reshape(input_shape[:-1])


# speedup vs baseline: 1.0690x; 1.0690x over previous
import functools

import jax
import jax.numpy as jnp
from jax import lax
from jax.experimental import pallas as pl
from jax.experimental.pallas import tpu as pltpu
from jax.experimental.pallas import tpu_sc as plsc

NUM_EMB = 1024
DIM = 64
N_TOKENS = 16 * 64 * 64
BN = 512
N_CHUNKS = 1
CHUNK = N_TOKENS // N_CHUNKS
GRID = CHUNK // BN

NC, NS = 2, 16
NW = NC * NS
B_PER_W = CHUNK // NW
CH = 128


def _dist_argmin_body(x_ref, w_ref, idx_ref, w2_ref, iotaf_ref, wm2_ref):
    @pl.when(pl.program_id(0) == 0)
    def _():
        w = w_ref[...]
        w2_ref[...] = jnp.sum(w * w, axis=1)[None, :]
        ii = lax.broadcasted_iota(jnp.int32, (1, NUM_EMB), 1)
        iotaf_ref[...] = ii.astype(jnp.float32)
        wm2_ref[...] = w * -2.0

    x = jnp.transpose(x_ref[0])
    x2 = jnp.sum(x * x, axis=1, keepdims=True)
    mm2 = lax.dot_general(x, wm2_ref[...], (((1,), (1,)), ((), ())),
                          preferred_element_type=jnp.float32)
    d = (x2 + w2_ref[...]) + mm2
    dmin = jnp.min(d, axis=1, keepdims=True)
    idxf = jnp.min(jnp.where(d == dmin, iotaf_ref[...], float(2 * NUM_EMB)),
                   axis=1)
    idx_ref[0, 0, :] = idxf.astype(jnp.int32)


def _encode(x_cm, W):
    sb = 4096 // BN
    return pl.pallas_call(
        _dist_argmin_body,
        grid=(GRID,),
        in_specs=[
            pl.BlockSpec((1, DIM, BN), lambda i: (i // sb, 0, i % sb)),
            pl.BlockSpec((NUM_EMB, DIM), lambda i: (0, 0)),
        ],
        out_specs=pl.BlockSpec((1, 1, BN), lambda i: (i, 0, 0)),
        out_shape=jax.ShapeDtypeStruct((GRID, 1, BN), jnp.int32),
        scratch_shapes=[pltpu.VMEM((1, NUM_EMB), jnp.float32),
                        pltpu.VMEM((1, NUM_EMB), jnp.float32),
                        pltpu.VMEM((NUM_EMB, DIM), jnp.float32)],
    )(x_cm, W)


@functools.partial(
    pl.kernel,
    out_type=jax.ShapeDtypeStruct((CHUNK, DIM), jnp.float32),
    mesh=plsc.VectorSubcoreMesh(core_axis_name="c", subcore_axis_name="s"),
    compiler_params=pltpu.CompilerParams(use_tc_tiling_on_sc=False),
    scratch_types=[
        pltpu.VMEM((B_PER_W,), jnp.int32),
        pltpu.VMEM((CH, DIM), jnp.float32),
        pltpu.SemaphoreType.DMA,
    ],
)
def _gather_rows(w_hbm, idx_hbm, out_hbm, idx_v, rows_v, sem):
    wid = lax.axis_index("s") * NC + lax.axis_index("c")
    base = wid * B_PER_W
    pltpu.sync_copy(idx_hbm.at[pl.ds(base, B_PER_W)], idx_v)
    for c in range(B_PER_W // CH):
        pltpu.async_copy(
            w_hbm.at[idx_v.at[pl.ds(c * CH, CH)]], rows_v, sem).wait()
        pltpu.sync_copy(rows_v, out_hbm.at[pl.ds(base + c * CH, CH)])


def kernel(inputs, W):
    b, c, h, w = inputs.shape
    x_cm = inputs.reshape(b, c, h * w)
    idx = _encode(x_cm, W).reshape(-1)
    quant_flat = _gather_rows(W, idx)
    quantized = quant_flat.reshape(b, h, w, c)
    quantized = jnp.transpose(quantized, (0, 3, 1, 2))
    return quantized, idx.reshape(b, h, w)


# speedup vs baseline: 1.3920x; 1.3022x over previous
import functools

import jax
import jax.numpy as jnp
from jax import lax
from jax.experimental import pallas as pl
from jax.experimental.pallas import tpu as pltpu
from jax.experimental.pallas import tpu_sc as plsc

NUM_EMB = 1024
DIM = 64
N_TOKENS = 16 * 64 * 64
BN = 4096
N_CHUNKS = 1
CHUNK = N_TOKENS // N_CHUNKS
GRID = CHUNK // BN

NC, NS = 2, 16
NW = NC * NS
B_PER_W = CHUNK // NW
CH = 128


def _dist_argmin_body(x_ref, w_ref, idx_ref, w2_ref, iotaf_ref, wm2_ref):
    @pl.when(pl.program_id(0) == 0)
    def _():
        w = w_ref[...]
        w2_ref[...] = jnp.sum(w * w, axis=1)[None, :]
        ii = lax.broadcasted_iota(jnp.int32, (1, NUM_EMB), 1)
        iotaf_ref[...] = ii.astype(jnp.float32)
        wm2_ref[...] = w * -2.0

    x = jnp.transpose(x_ref[0])
    x2 = jnp.sum(x * x, axis=1, keepdims=True)
    mm2 = lax.dot_general(x, wm2_ref[...], (((1,), (1,)), ((), ())),
                          preferred_element_type=jnp.float32)
    d = (x2 + w2_ref[...]) + mm2
    idx_ref[0, 0, :] = jnp.argmin(d, axis=1).astype(jnp.int32)


def _encode(x_cm, W):
    sb = 4096 // BN
    return pl.pallas_call(
        _dist_argmin_body,
        grid=(GRID,),
        in_specs=[
            pl.BlockSpec((1, DIM, BN), lambda i: (i // sb, 0, i % sb)),
            pl.BlockSpec((NUM_EMB, DIM), lambda i: (0, 0)),
        ],
        out_specs=pl.BlockSpec((1, 1, BN), lambda i: (i, 0, 0)),
        out_shape=jax.ShapeDtypeStruct((GRID, 1, BN), jnp.int32),
        scratch_shapes=[pltpu.VMEM((1, NUM_EMB), jnp.float32),
                        pltpu.VMEM((1, NUM_EMB), jnp.float32),
                        pltpu.VMEM((NUM_EMB, DIM), jnp.float32)],
    )(x_cm, W)


@functools.partial(
    pl.kernel,
    out_type=jax.ShapeDtypeStruct((CHUNK, DIM), jnp.float32),
    mesh=plsc.VectorSubcoreMesh(core_axis_name="c", subcore_axis_name="s"),
    compiler_params=pltpu.CompilerParams(use_tc_tiling_on_sc=False),
    scratch_types=[
        pltpu.VMEM((B_PER_W,), jnp.int32),
        pltpu.VMEM((CH, DIM), jnp.float32),
        pltpu.SemaphoreType.DMA,
    ],
)
def _gather_rows(w_hbm, idx_hbm, out_hbm, idx_v, rows_v, sem):
    wid = lax.axis_index("s") * NC + lax.axis_index("c")
    base = wid * B_PER_W
    pltpu.sync_copy(idx_hbm.at[pl.ds(base, B_PER_W)], idx_v)
    for c in range(B_PER_W // CH):
        pltpu.async_copy(
            w_hbm.at[idx_v.at[pl.ds(c * CH, CH)]], rows_v, sem).wait()
        pltpu.sync_copy(rows_v, out_hbm.at[pl.ds(base + c * CH, CH)])


def kernel(inputs, W):
    b, c, h, w = inputs.shape
    x_cm = inputs.reshape(b, c, h * w)
    idx = _encode(x_cm, W).reshape(-1)
    quant_flat = _gather_rows(W, idx)
    quantized = quant_flat.reshape(b, h, w, c)
    quantized = jnp.transpose(quantized, (0, 3, 1, 2))
    return quantized, idx.reshape(b, h, w)
